# baseline (speedup 1.0000x reference)
import jax
import jax.numpy as jnp
from jax import lax
from jax.experimental import pallas as pl
from jax.experimental.pallas import tpu as pltpu
from jax.experimental.pallas import tpu_sc as plsc

N_NODES = 10000
N_EDGES = 320000
D = 128
DH = D // 2
BN_EPS = 1e-5

NC = 2
NS = 16
EPW = N_EDGES // NS
CHUNK = 400
NCHUNK = EPW // CHUNK
NBUF = 3
NFULL = (NCHUNK // NBUF) * NBUF
ROWS_PER_SUB = 624
TAIL_ROWS = N_NODES - NS * ROWS_PER_SUB
TAIL_BASE = NS * ROWS_PER_SUB


def _sc_aggregate_body(xh_hbm, sx_hbm, ei_hbm, ew_hbm, out_hbm,
                       src0, src1, src2, dst0, dst1, dst2, ew0, ew1, ew2,
                       rows0, rows1, rows2, acc,
                       psem0, psem1, psem2, isem0, isem1, isem2,
                       wsem0, wsem1, wsem2, gsem0, gsem1, gsem2,
                       ssem0, ssem1, ssem2):
  c = lax.axis_index("c")
  s = lax.axis_index("s")
  rows = (rows0, rows1, rows2)
  srcb = (src0, src1, src2)
  dstb = (dst0, dst1, dst2)
  ewb = (ew0, ew1, ew2)
  psem = (psem0, psem1, psem2)
  isem = (isem0, isem1, isem2)
  wsem = (wsem0, wsem1, wsem2)
  gsem = (gsem0, gsem1, gsem2)
  ssem = (ssem0, ssem1, ssem2)
  ebase = s * EPW
  r0 = s * ROWS_PER_SUB

  iota16 = lax.iota(jnp.int32, 16)

  def _init_piece(row_off, n):
    for g in range(n // 16 + (1 if n % 16 else 0)):
      base = r0 + row_off + g * 16
      src0[pl.ds(g * 16, 16)] = 2 * (base + iota16) + c
    pltpu.async_copy(xh_hbm.at[src0.at[pl.ds(0, n)]],
                     rows0.at[pl.ds(0, n)], gsem0).wait()
    pltpu.sync_copy(rows0.at[pl.ds(0, n)], acc.at[pl.ds(r0 + row_off, n)])

  _init_piece(0, CHUNK)
  _init_piece(CHUNK, ROWS_PER_SUB - CHUNK)

  @pl.when(s == NS - 1)
  def _init_tail():
    src0[pl.ds(0, 16)] = 2 * (TAIL_BASE + iota16) + c
    pltpu.async_copy(xh_hbm.at[src0.at[pl.ds(0, TAIL_ROWS)]],
                     rows0.at[pl.ds(0, TAIL_ROWS)], gsem0).wait()
    pltpu.sync_copy(rows0.at[pl.ds(0, TAIL_ROWS)],
                    acc.at[pl.ds(TAIL_BASE, TAIL_ROWS)])

  plsc.subcore_barrier()

  def _stage_src(k, b):
    pltpu.async_copy(sx_hbm.at[c, pl.ds(ebase + k * CHUNK, CHUNK)],
                     srcb[b], psem[b])

  def _wait_src(k, b):
    pltpu.make_async_copy(sx_hbm.at[c, pl.ds(ebase + k * CHUNK, CHUNK)],
                          srcb[b], psem[b]).wait()

  def _stage_de(k, b):
    off = ebase + k * CHUNK
    pltpu.async_copy(ei_hbm.at[1, pl.ds(off, CHUNK)], dstb[b], isem[b])
    pltpu.async_copy(ew_hbm.at[pl.ds(off, CHUNK)], ewb[b], wsem[b])

  def _wait_dst(k, b):
    pltpu.make_async_copy(ei_hbm.at[1, pl.ds(ebase + k * CHUNK, CHUNK)],
                          dstb[b], isem[b]).wait()

  def _wait_ew(k, b):
    pltpu.make_async_copy(ew_hbm.at[pl.ds(ebase + k * CHUNK, CHUNK)],
                          ewb[b], wsem[b]).wait()

  def _issue_gather(b):
    pltpu.async_copy(xh_hbm.at[srcb[b]], rows[b], gsem[b])

  def _wait_gather(b):
    pltpu.make_async_copy(xh_hbm.at[srcb[b]], rows[b], gsem[b]).wait()

  def _wait_scatter(b):
    pltpu.make_async_copy(rows[b], acc.at[dstb[b]], ssem[b]).wait()

  def _step(k, b):
    nb = (b + 1) % NBUF
    nnb = (b + 2) % NBUF

    @pl.when(k >= NBUF - 1)
    def _free_next():
      _wait_scatter(nb)

    @pl.when(k + 1 < NCHUNK)
    def _issue_next():
      _wait_src(k + 1, nb)
      _issue_gather(nb)
      _stage_de(k + 1, nb)

    @pl.when(k + 2 < NCHUNK)
    def _stage_ahead():
      _stage_src(k + 2, nnb)

    _wait_gather(b)
    _wait_ew(k, b)

    @plsc.parallel_loop(0, CHUNK // 16, unroll=4)
    def _scale(g):
      w16 = ewb[b][pl.ds(g * 16, 16)]
      for i in range(16):
        wv = lax.gather(
            w16, jnp.full((16, 1), i, jnp.int32),
            lax.GatherDimensionNumbers(offset_dims=(),
                                       collapsed_slice_dims=(0,),
                                       start_index_map=(0,)),
            (1,), mode=lax.GatherScatterMode.PROMISE_IN_BOUNDS)
        e = g * 16 + i
        for j in range(DH // 16):
          rb = rows[b]
          rb[e, pl.ds(j * 16, 16)] = rb[e, pl.ds(j * 16, 16)] * wv

    _wait_dst(k, b)
    pltpu.async_copy(rows[b], acc.at[dstb[b]], ssem[b], add=True)

  _stage_src(0, 0)
  _stage_src(1, 1)
  _wait_src(0, 0)
  _issue_gather(0)
  _stage_de(0, 0)

  def _outer(t, carry):
    for b in range(NBUF):
      _step(t * NBUF + b, b)
    return carry
  lax.fori_loop(0, NFULL // NBUF, _outer, 0)

  for k in range(NFULL, NCHUNK):
    _step(jnp.int32(k), k % NBUF)

  for k in range(NCHUNK - NBUF + 1, NCHUNK):
    _wait_scatter(k % NBUF)

  plsc.subcore_barrier()
  pltpu.sync_copy(acc.at[pl.ds(r0, ROWS_PER_SUB)],
                  out_hbm.at[c, pl.ds(r0, ROWS_PER_SUB)])

  @pl.when(s == NS - 1)
  def _flush_tail():
    pltpu.sync_copy(acc.at[pl.ds(TAIL_BASE, TAIL_ROWS)],
                    out_hbm.at[c, pl.ds(TAIL_BASE, TAIL_ROWS)])


@jax.jit
def _sc_aggregate(x_halfrows, src_x2, edge_index, ew):
  mesh = plsc.VectorSubcoreMesh(core_axis_name="c", subcore_axis_name="s")
  return pl.kernel(
      _sc_aggregate_body,
      out_type=jax.ShapeDtypeStruct((NC, N_NODES, DH), jnp.float32),
      mesh=mesh,
      compiler_params=pltpu.CompilerParams(use_tc_tiling_on_sc=False),
      scratch_types=[
          pltpu.VMEM((CHUNK,), jnp.int32),
          pltpu.VMEM((CHUNK,), jnp.int32),
          pltpu.VMEM((CHUNK,), jnp.int32),
          pltpu.VMEM((CHUNK,), jnp.int32),
          pltpu.VMEM((CHUNK,), jnp.int32),
          pltpu.VMEM((CHUNK,), jnp.int32),
          pltpu.VMEM((CHUNK,), jnp.float32),
          pltpu.VMEM((CHUNK,), jnp.float32),
          pltpu.VMEM((CHUNK,), jnp.float32),
          pltpu.VMEM((CHUNK, DH), jnp.float32),
          pltpu.VMEM((CHUNK, DH), jnp.float32),
          pltpu.VMEM((CHUNK, DH), jnp.float32),
          pltpu.VMEM_SHARED((N_NODES, DH), jnp.float32),
      ] + [pltpu.SemaphoreType.DMA] * 15,
  )(x_halfrows, src_x2, edge_index, ew)


def _bn_relu(y, g, be):
  m = jnp.mean(y, axis=0, keepdims=True)
  v = jnp.mean((y - m) ** 2, axis=0, keepdims=True)
  return jnp.maximum(g * (y - m) * lax.rsqrt(v + BN_EPS) + be, 0.0)


def _tc_mlp_body(a0_ref, a1_ref, W1a_ref, W1b_ref, b1_ref, g1_ref, be1_ref,
                 W2_ref, b2_ref, g2_ref, be2_ref, g3_ref, be3_ref, out_ref):
  y = (jnp.dot(a0_ref[...], W1a_ref[...], preferred_element_type=jnp.float32)
       + jnp.dot(a1_ref[...], W1b_ref[...], preferred_element_type=jnp.float32)
       + b1_ref[...])
  h = _bn_relu(y, g1_ref[...], be1_ref[...])
  y = jnp.dot(h, W2_ref[...], preferred_element_type=jnp.float32) + b2_ref[...]
  h = _bn_relu(y, g2_ref[...], be2_ref[...])
  out_ref[...] = _bn_relu(h, g3_ref[...], be3_ref[...])


def kernel(x, edge_index, edge_weight, W1, b1, g1, be1, W2, b2, g2, be2,
           g3, be3):
  x_halfrows = x.reshape(2 * N_NODES, DH)
  ei = edge_index.astype(jnp.int32)
  s2 = ei[0] * 2
  src_x2 = jnp.stack([s2, s2 + 1])
  h = _sc_aggregate(x_halfrows, src_x2, ei, edge_weight)

  r = lambda v: v.reshape(1, D)
  return pl.pallas_call(
      _tc_mlp_body,
      out_shape=jax.ShapeDtypeStruct((N_NODES, D), jnp.float32),
  )(h[0], h[1], W1[:DH], W1[DH:], r(b1), r(g1), r(be1),
    W2, r(b2), r(g2), r(be2), r(g3), r(be3))


# speedup vs baseline: 1.0847x; 1.0847x over previous
import jax
import jax.numpy as jnp
from jax import lax
from jax.experimental import pallas as pl
from jax.experimental.pallas import tpu as pltpu
from jax.experimental.pallas import tpu_sc as plsc

N_NODES = 10000
N_EDGES = 320000
D = 128
DH = D // 2
BN_EPS = 1e-5

NC = 2
NS = 16
EPW = N_EDGES // NS
CHUNK = 200
CPAD = 208
NCHUNK = EPW // CHUNK
NBUF = 3
NFULL = (NCHUNK // NBUF) * NBUF
ROWS_PER_SUB = 624
TAIL_ROWS = N_NODES - NS * ROWS_PER_SUB
TAIL_BASE = NS * ROWS_PER_SUB


def _sc_aggregate_body(xh_hbm, ei_hbm, ew_hbm, out_hbm,
                       src_all, dst0, dst1, dst2, ew0, ew1, ew2,
                       rows0, rows1, rows2, acc,
                       gsem0, gsem1, gsem2, isem0, isem1, isem2,
                       wsem0, wsem1, wsem2, ssem0, ssem1, ssem2):
  c = lax.axis_index("c")
  s = lax.axis_index("s")
  rows = (rows0, rows1, rows2)
  dstb = (dst0, dst1, dst2)
  ewb = (ew0, ew1, ew2)
  gsem = (gsem0, gsem1, gsem2)
  isem = (isem0, isem1, isem2)
  wsem = (wsem0, wsem1, wsem2)
  ssem = (ssem0, ssem1, ssem2)
  ebase = s * EPW
  r0 = s * ROWS_PER_SUB

  iota16 = lax.iota(jnp.int32, 16)

  @plsc.parallel_loop(0, 40, unroll=4)
  def _init_idx(g):
    base = r0 + g * 16
    src_all[pl.ds(g * 16, 16)] = 2 * (base + iota16) + c

  @pl.when(s == NS - 1)
  def _tail_idx():
    src_all[pl.ds(ROWS_PER_SUB, TAIL_ROWS)] = 2 * (TAIL_BASE + iota16) + c

  off = 0
  while off < ROWS_PER_SUB:
    p = min(CHUNK, ROWS_PER_SUB - off)
    pltpu.async_copy(xh_hbm.at[src_all.at[pl.ds(off, p)]],
                     rows0.at[pl.ds(0, p)], gsem0).wait()
    pltpu.sync_copy(rows0.at[pl.ds(0, p)], acc.at[pl.ds(r0 + off, p)])
    off += p

  @pl.when(s == NS - 1)
  def _init_tail():
    pltpu.async_copy(
        xh_hbm.at[src_all.at[pl.ds(ROWS_PER_SUB, TAIL_ROWS)]],
        rows2.at[pl.ds(0, TAIL_ROWS)], gsem2).wait()
    pltpu.sync_copy(rows2.at[pl.ds(0, TAIL_ROWS)],
                    acc.at[pl.ds(TAIL_BASE, TAIL_ROWS)])

  pltpu.sync_copy(ei_hbm.at[0, pl.ds(ebase, EPW)], src_all)

  @plsc.parallel_loop(0, EPW // 16, unroll=4)
  def _xform(g):
    v = src_all[pl.ds(g * 16, 16)]
    src_all[pl.ds(g * 16, 16)] = v + v + c

  plsc.subcore_barrier()

  def _issue(k, b):
    off = ebase + k * CHUNK
    pltpu.async_copy(
        xh_hbm.at[src_all.at[pl.ds(k * CHUNK, CHUNK)]],
        rows[b].at[pl.ds(0, CHUNK)], gsem[b])
    pltpu.async_copy(ei_hbm.at[1, pl.ds(off, CHUNK)], dstb[b], isem[b])
    pltpu.async_copy(ew_hbm.at[pl.ds(off, CHUNK)],
                     ewb[b].at[pl.ds(0, CHUNK)], wsem[b])

  def _wait_in(k, b):
    off = ebase + k * CHUNK
    pltpu.make_async_copy(
        xh_hbm.at[src_all.at[pl.ds(k * CHUNK, CHUNK)]],
        rows[b].at[pl.ds(0, CHUNK)], gsem[b]).wait()
    pltpu.make_async_copy(ew_hbm.at[pl.ds(off, CHUNK)],
                          ewb[b].at[pl.ds(0, CHUNK)], wsem[b]).wait()

  def _wait_dst(k, b):
    off = ebase + k * CHUNK
    pltpu.make_async_copy(ei_hbm.at[1, pl.ds(off, CHUNK)], dstb[b],
                          isem[b]).wait()

  def _wait_scatter(b):
    pltpu.make_async_copy(rows[b].at[pl.ds(0, CHUNK)], acc.at[dstb[b]],
                          ssem[b]).wait()

  def _step(k, b):
    nb = (b + 1) % NBUF

    @pl.when(k >= NBUF - 1)
    def _free_next():
      _wait_scatter(nb)

    @pl.when(k + 1 < NCHUNK)
    def _issue_next():
      _issue(k + 1, nb)

    _wait_in(k, b)

    @plsc.parallel_loop(0, CPAD // 16, unroll=4)
    def _scale(g):
      w16 = ewb[b][pl.ds(g * 16, 16)]
      for i in range(16):
        wv = lax.gather(
            w16, jnp.full((16, 1), i, jnp.int32),
            lax.GatherDimensionNumbers(offset_dims=(),
                                       collapsed_slice_dims=(0,),
                                       start_index_map=(0,)),
            (1,), mode=lax.GatherScatterMode.PROMISE_IN_BOUNDS)
        e = g * 16 + i
        for j in range(DH // 16):
          rb = rows[b]
          rb[e, pl.ds(j * 16, 16)] = rb[e, pl.ds(j * 16, 16)] * wv

    _wait_dst(k, b)
    pltpu.async_copy(rows[b].at[pl.ds(0, CHUNK)], acc.at[dstb[b]],
                     ssem[b], add=True)

  _issue(0, 0)

  def _outer(t, carry):
    for b in range(NBUF):
      _step(t * NBUF + b, b)
    return carry
  lax.fori_loop(0, NFULL // NBUF, _outer, 0)

  for k in range(NFULL, NCHUNK):
    _step(jnp.int32(k), k % NBUF)

  for k in range(NCHUNK - NBUF + 1, NCHUNK):
    _wait_scatter(k % NBUF)

  plsc.subcore_barrier()
  pltpu.sync_copy(acc.at[pl.ds(r0, ROWS_PER_SUB)],
                  out_hbm.at[c, pl.ds(r0, ROWS_PER_SUB)])

  @pl.when(s == NS - 1)
  def _flush_tail():
    pltpu.sync_copy(acc.at[pl.ds(TAIL_BASE, TAIL_ROWS)],
                    out_hbm.at[c, pl.ds(TAIL_BASE, TAIL_ROWS)])


@jax.jit
def _sc_aggregate(x_halfrows, edge_index, ew):
  mesh = plsc.VectorSubcoreMesh(core_axis_name="c", subcore_axis_name="s")
  return pl.kernel(
      _sc_aggregate_body,
      out_type=jax.ShapeDtypeStruct((NC, N_NODES, DH), jnp.float32),
      mesh=mesh,
      compiler_params=pltpu.CompilerParams(use_tc_tiling_on_sc=False),
      scratch_types=[
          pltpu.VMEM((EPW,), jnp.int32),
          pltpu.VMEM((CHUNK,), jnp.int32),
          pltpu.VMEM((CHUNK,), jnp.int32),
          pltpu.VMEM((CHUNK,), jnp.int32),
          pltpu.VMEM((CPAD,), jnp.float32),
          pltpu.VMEM((CPAD,), jnp.float32),
          pltpu.VMEM((CPAD,), jnp.float32),
          pltpu.VMEM((CPAD, DH), jnp.float32),
          pltpu.VMEM((CPAD, DH), jnp.float32),
          pltpu.VMEM((CPAD, DH), jnp.float32),
          pltpu.VMEM_SHARED((N_NODES, DH), jnp.float32),
      ] + [pltpu.SemaphoreType.DMA] * 12,
  )(x_halfrows, edge_index, ew)


def _bn_relu(y, g, be):
  m = jnp.mean(y, axis=0, keepdims=True)
  v = jnp.mean((y - m) ** 2, axis=0, keepdims=True)
  return jnp.maximum(g * (y - m) * lax.rsqrt(v + BN_EPS) + be, 0.0)


def _tc_mlp_body(a0_ref, a1_ref, W1a_ref, W1b_ref, b1_ref, g1_ref, be1_ref,
                 W2_ref, b2_ref, g2_ref, be2_ref, g3_ref, be3_ref, out_ref):
  y = (jnp.dot(a0_ref[...], W1a_ref[...], preferred_element_type=jnp.float32)
       + jnp.dot(a1_ref[...], W1b_ref[...], preferred_element_type=jnp.float32)
       + b1_ref[...])
  h = _bn_relu(y, g1_ref[...], be1_ref[...])
  y = jnp.dot(h, W2_ref[...], preferred_element_type=jnp.float32) + b2_ref[...]
  h = _bn_relu(y, g2_ref[...], be2_ref[...])
  out_ref[...] = _bn_relu(h, g3_ref[...], be3_ref[...])


def kernel(x, edge_index, edge_weight, W1, b1, g1, be1, W2, b2, g2, be2,
           g3, be3):
  x_halfrows = x.reshape(2 * N_NODES, DH)
  h = _sc_aggregate(x_halfrows, edge_index.astype(jnp.int32), edge_weight)

  r = lambda v: v.reshape(1, D)
  return pl.pallas_call(
      _tc_mlp_body,
      out_shape=jax.ShapeDtypeStruct((N_NODES, D), jnp.float32),
  )(h[0], h[1], W1[:DH], W1[DH:], r(b1), r(g1), r(be1),
    W2, r(b2), r(g2), r(be2), r(g3), r(be3))
